# baseline (speedup 1.0000x reference)
import functools

import jax
import jax.numpy as jnp
from jax import lax
from jax.experimental import pallas as pl
from jax.experimental.pallas import tpu as pltpu
from jax.experimental.pallas import tpu_sc as plsc

F32 = jnp.float32
I32 = jnp.int32

N = 50000
NP = 50176
H = 128
WA = 144
E = 300000
EPT = 18752
EP = EPT * 16
L = 100000
LP = 102400
HW = 16
CHUNK = 6272
NCHUNK = NP // CHUNK
ACC_ROWS = CHUNK + 16
SPAN = CHUNK // 16
LPT = LP // 32

_mesh = plsc.VectorSubcoreMesh(
    core_axis_name="c", subcore_axis_name="s", num_cores=2, num_subcores=16)
_sc_params = pltpu.CompilerParams(needs_layout_passes=False,
                                  use_tc_tiling_on_sc=False)


@functools.partial(
    pl.kernel,
    out_type=(jax.ShapeDtypeStruct((NP, WA), F32),
              jax.ShapeDtypeStruct((NP, WA), F32)),
    mesh=_mesh,
    scratch_types=[
        pltpu.VMEM((EPT + 128,), I32),
        pltpu.VMEM((2, 128, WA), F32),
        pltpu.VMEM((2, 128), I32),
        pltpu.VMEM((3, 128), I32),
        pltpu.VMEM((3, 128), I32),
        pltpu.VMEM((56, WA), F32),
        pltpu.VMEM_SHARED((ACC_ROWS, WA), F32),
        pltpu.SemaphoreType.DMA,
        pltpu.SemaphoreType.DMA,
        pltpu.SemaphoreType.DMA,
    ],
    compiler_params=_sc_params,
)
def _agg_pair(srcA, dstA, tabA, srcB, dstB, tabB, outA, outB,
              dst_buf, rows, vbuf, pv, dv, zbuf, acc, sem_v, sem_g, sem_s):
    c = lax.axis_index("c")
    s = lax.axis_index("s")
    zvec = jnp.zeros((16,), F32)

    def _zb(i, carry):
        for k in range(9):
            zbuf[i, pl.ds(k * 16, 16)] = zvec
        return carry
    lax.fori_loop(0, 56, _zb, 0)

    io = lax.iota(I32, 16)

    _dnums = lax.GatherDimensionNumbers(
        offset_dims=(), collapsed_slice_dims=(0,), start_index_map=(0,))

    def _permute(x, idx):
        return lax.gather(x, idx[:, None], _dnums, slice_sizes=(1,),
                          mode=lax.GatherScatterMode.PROMISE_IN_BOUNDS)

    def _prefix(m):
        x = jnp.where(m, 1, 0).astype(I32)
        for k in (1, 2, 4, 8):
            g = _permute(x, jnp.maximum(io - k, 0))
            x = x + jnp.where(io >= k, g, 0)
        return x

    def _process(src_hbm, dst_hbm, table_hbm, out_hbm):
        def _unpack(slot, b):
            for k in range(8):
                v = dst_buf[pl.ds(b * 128 + k * 16, 16)]
                dv[slot, pl.ds(k * 16, 16)] = v & 8191
                pv[slot, pl.ds(k * 16, 16)] = (v >> 13) + s * EPT

        def _val_gather(slot, vslot):
            pltpu.async_copy(src_hbm.at[pv.at[slot]], vbuf.at[vslot], sem_v)

        def chunk_body(p, carry):
            lo = p * CHUNK
            for k in range(7):
                pltpu.sync_copy(zbuf, acc.at[pl.ds(s * SPAN + k * 56, 56), :])
            pltpu.sync_copy(dst_hbm.at[pl.ds(s * EPT, EPT)],
                            dst_buf.at[pl.ds(0, EPT)])
            plsc.subcore_barrier()

            def scan_body(i, ptr):
                d = dst_buf[pl.ds(i * 16, 16)]
                m = (d >= lo) & (d < lo + CHUNK)
                inc = _prefix(m)
                tgt = ptr + inc - 1
                packed = (d - lo) | ((i * 16 + io) << 13)
                plsc.store_scatter(dst_buf, [tgt], packed, mask=m)
                return ptr + plsc.all_reduce_population_count(m)
            ptr = lax.fori_loop(0, EPT // 16, scan_body,
                                jnp.zeros((16,), I32))
            for k in range(8):
                tgt = ptr + k * 16 + io
                plsc.store_scatter(dst_buf, [tgt], CHUNK + io)
            nb = (ptr[0] + 127) // 128

            @pl.when(nb > 0)
            def _prolog():
                _unpack(0, 0)
                _val_gather(0, 0)

            def batch_body(b, carry2):
                jm2 = b % 2
                jm3 = b % 3
                pltpu.make_async_copy(src_hbm.at[pv.at[jm3]],
                                      vbuf.at[jm2], sem_v).wait()

                @pl.when(b + 1 < nb)
                def _prefetch():
                    _unpack((b + 1) % 3, b + 1)
                    _val_gather((b + 1) % 3, (b + 1) % 2)

                pltpu.async_copy(table_hbm.at[vbuf.at[jm2]],
                                 rows.at[jm2], sem_g).wait()

                @pl.when(b > 0)
                def _drain():
                    pltpu.make_async_copy(rows.at[jm2],
                                          acc.at[dv.at[jm3]], sem_s).wait()
                pltpu.async_copy(rows.at[jm2], acc.at[dv.at[jm3]], sem_s,
                                 add=True)
                return carry2
            lax.fori_loop(0, nb, batch_body, 0)

            @pl.when(nb > 0)
            def _epilog():
                pltpu.make_async_copy(rows.at[0], acc.at[dv.at[0]],
                                      sem_s).wait()
            plsc.subcore_barrier()
            pltpu.sync_copy(acc.at[pl.ds(s * SPAN, SPAN), :],
                            out_hbm.at[pl.ds(lo + s * SPAN, SPAN), :])
            plsc.subcore_barrier()
            return carry
        lax.fori_loop(0, NCHUNK, chunk_body, 0)

    @pl.when(c == 0)
    def _type_a():
        _process(srcA, dstA, tabA, outA)

    @pl.when(c == 1)
    def _type_b():
        _process(srcB, dstB, tabB, outB)


@functools.partial(
    pl.kernel,
    out_type=jax.ShapeDtypeStruct((LP, HW), F32),
    mesh=_mesh,
    scratch_types=[
        pltpu.VMEM((LPT,), I32),
        pltpu.VMEM((LPT,), I32),
        pltpu.VMEM((128, HW), F32),
        pltpu.SemaphoreType.DMA,
    ],
    compiler_params=_sc_params,
)
def _head(i0_hbm, i1_hbm, gm_hbm, gv_hbm, out_hbm, i0_buf, i1_buf, ra, sem):
    c = lax.axis_index("c")
    s = lax.axis_index("s")
    w = s * 2 + c
    base = w * LPT
    pltpu.sync_copy(i0_hbm.at[pl.ds(base, LPT)], i0_buf)
    pltpu.sync_copy(i1_hbm.at[pl.ds(base, LPT)], i1_buf)

    def body(b, carry):
        pltpu.async_copy(gm_hbm.at[i0_buf.at[pl.ds(b * 128, 128)]],
                         ra, sem).wait()
        pltpu.async_copy(gv_hbm.at[i1_buf.at[pl.ds(b * 128, 128)]],
                         ra, sem, add=True).wait()
        pltpu.sync_copy(ra, out_hbm.at[pl.ds(base + b * 128, 128), :])
        return carry
    lax.fori_loop(0, LPT // 128, body, 0)


def _flag_cols(nrows):
    return (lax.broadcasted_iota(I32, (nrows, 16), 1) == 0).astype(F32)


def _proj_body(x_ref, w_ref, b_ref, o_ref):
    h = jnp.dot(x_ref[...], w_ref[...], preferred_element_type=F32) + b_ref[...]
    o_ref[...] = jnp.concatenate([h, _flag_cols(h.shape[0])], axis=1)


_proj = pl.pallas_call(
    _proj_body,
    grid=(NP // SPAN,),
    in_specs=[pl.BlockSpec((SPAN, H), lambda i: (i, 0)),
              pl.BlockSpec((H, H), lambda i: (0, 0)),
              pl.BlockSpec((1, H), lambda i: (0, 0))],
    out_specs=pl.BlockSpec((SPAN, WA), lambda i: (i, 0)),
    out_shape=jax.ShapeDtypeStruct((NP, WA), F32),
)


def _mean_h(agg_ref, xd_ref, wl_ref, bl_ref, wr_ref):
    aggv = agg_ref[:, :H]
    cnt = agg_ref[:, H:H + 1]
    mean = aggv / jnp.maximum(cnt, 1.0)
    h = (jnp.dot(mean, wl_ref[...], preferred_element_type=F32) + bl_ref[...]
         + jnp.dot(xd_ref[:, :H], wr_ref[...], preferred_element_type=F32))
    return jnp.maximum(h, 0.0)


def _comb1_body(agg_ref, xd_ref, wl_ref, bl_ref, wr_ref, o_ref):
    h = _mean_h(agg_ref, xd_ref, wl_ref, bl_ref, wr_ref)
    o_ref[...] = jnp.concatenate([h, _flag_cols(h.shape[0])], axis=1)


_comb1 = pl.pallas_call(
    _comb1_body,
    grid=(NP // SPAN,),
    in_specs=[pl.BlockSpec((SPAN, WA), lambda i: (i, 0)),
              pl.BlockSpec((SPAN, WA), lambda i: (i, 0)),
              pl.BlockSpec((H, H), lambda i: (0, 0)),
              pl.BlockSpec((1, H), lambda i: (0, 0)),
              pl.BlockSpec((H, H), lambda i: (0, 0))],
    out_specs=pl.BlockSpec((SPAN, WA), lambda i: (i, 0)),
    out_shape=jax.ShapeDtypeStruct((NP, WA), F32),
)


def _comb2_body(agg_ref, xd_ref, wl_ref, bl_ref, wr_ref, wc_ref, bc_ref, o_ref):
    h = _mean_h(agg_ref, xd_ref, wl_ref, bl_ref, wr_ref)
    o_ref[...] = jnp.dot(h, wc_ref[...], preferred_element_type=F32) + bc_ref[...]


_comb2 = pl.pallas_call(
    _comb2_body,
    grid=(NP // SPAN,),
    in_specs=[pl.BlockSpec((SPAN, WA), lambda i: (i, 0)),
              pl.BlockSpec((SPAN, WA), lambda i: (i, 0)),
              pl.BlockSpec((H, H), lambda i: (0, 0)),
              pl.BlockSpec((1, H), lambda i: (0, 0)),
              pl.BlockSpec((H, H), lambda i: (0, 0)),
              pl.BlockSpec((H, HW), lambda i: (0, 0)),
              pl.BlockSpec((1, HW), lambda i: (0, 0))],
    out_specs=pl.BlockSpec((SPAN, HW), lambda i: (i, 0)),
    out_shape=jax.ShapeDtypeStruct((NP, HW), F32),
)


def _pad_edges(ei):
    pad = EP - E
    src = jnp.concatenate([ei[0], jnp.zeros((pad,), I32)])
    dst = jnp.concatenate([ei[1], jnp.full((pad,), 1 << 20, I32)])
    return src, dst


def kernel(x_mouse, x_virus, W_mouse, b_mouse, W_virus, b_virus,
           c1mv_Wl, c1mv_bl, c1mv_Wr, c1vm_Wl, c1vm_bl, c1vm_Wr,
           c2mv_Wl, c2mv_bl, c2mv_Wr, c2vm_Wl, c2vm_bl, c2vm_Wr,
           W_cls, b_cls, edge_index_mv, edge_index_vm, edge_label_index):
    src_mv, dst_mv = _pad_edges(edge_index_mv)
    src_vm, dst_vm = _pad_edges(edge_index_vm)

    hm0 = _proj(x_mouse, W_mouse, b_mouse[None])
    hv0 = _proj(x_virus, W_virus, b_virus[None])

    aggv1, aggm1 = _agg_pair(src_mv, dst_mv, hm0, src_vm, dst_vm, hv0)
    hv1 = _comb1(aggv1, hv0, c1mv_Wl, c1mv_bl[None], c1mv_Wr)
    hm1 = _comb1(aggm1, hm0, c1vm_Wl, c1vm_bl[None], c1vm_Wr)

    aggv2, aggm2 = _agg_pair(src_mv, dst_mv, hm1, src_vm, dst_vm, hv1)

    wc_m = jnp.pad(W_cls[:H], ((0, 0), (0, HW - 2)))
    wc_v = jnp.pad(W_cls[H:], ((0, 0), (0, HW - 2)))
    bc = jnp.pad(b_cls, (0, HW - 2))[None]
    gv = _comb2(aggv2, hv1, c2mv_Wl, c2mv_bl[None], c2mv_Wr, wc_v,
                jnp.zeros((1, HW), F32))
    gm = _comb2(aggm2, hm1, c2vm_Wl, c2vm_bl[None], c2vm_Wr, wc_m, bc)

    i0 = jnp.concatenate([edge_label_index[0], jnp.zeros((LP - L,), I32)])
    i1 = jnp.concatenate([edge_label_index[1], jnp.zeros((LP - L,), I32)])
    out = _head(i0, i1, gm, gv)
    return out[:L, :2]


# speedup vs baseline: 1.3845x; 1.3845x over previous
import functools

import jax
import jax.numpy as jnp
from jax import lax
from jax.experimental import pallas as pl
from jax.experimental.pallas import tpu as pltpu
from jax.experimental.pallas import tpu_sc as plsc

F32 = jnp.float32
I32 = jnp.int32

N = 50000
NP = 50176
H = 128
WA = 144
E = 300000
EPT = 18752
EP = EPT * 16
L = 100000
LP = 102400
HW = 16
CHUNK = 6272
NCHUNK = NP // CHUNK
ACC_ROWS = CHUNK + 16
SPAN = CHUNK // 16
LPT = LP // 32

_mesh = plsc.VectorSubcoreMesh(
    core_axis_name="c", subcore_axis_name="s", num_cores=2, num_subcores=16)
_sc_params = pltpu.CompilerParams(needs_layout_passes=False,
                                  use_tc_tiling_on_sc=False)


@functools.partial(
    pl.kernel,
    out_type=jax.ShapeDtypeStruct((NP, WA), F32),
    mesh=_mesh,
    scratch_types=[
        pltpu.VMEM((EPT + 128,), I32),
        pltpu.VMEM((2, 128, WA), F32),
        pltpu.VMEM((2, 128), I32),
        pltpu.VMEM((3, 128), I32),
        pltpu.VMEM((3, 128), I32),
        pltpu.VMEM((56, WA), F32),
        pltpu.VMEM_SHARED((ACC_ROWS, WA), F32),
        pltpu.SemaphoreType.DMA,
        pltpu.SemaphoreType.DMA,
        pltpu.SemaphoreType.DMA,
    ],
    compiler_params=_sc_params,
)
def _aggregate(src_hbm0, dst_hbm0, table_hbm0, out_hbm0,
               dst_buf, rows, vbuf, pv, dv, zbuf, acc, sem_v, sem_g, sem_s):
    c = lax.axis_index("c")
    s = lax.axis_index("s")
    zvec = jnp.zeros((16,), F32)

    def _zb(i, carry):
        for k in range(9):
            zbuf[i, pl.ds(k * 16, 16)] = zvec
        return carry
    lax.fori_loop(0, 56, _zb, 0)

    io = lax.iota(I32, 16)

    _dnums = lax.GatherDimensionNumbers(
        offset_dims=(), collapsed_slice_dims=(0,), start_index_map=(0,))

    def _permute(x, idx):
        return lax.gather(x, idx[:, None], _dnums, slice_sizes=(1,),
                          mode=lax.GatherScatterMode.PROMISE_IN_BOUNDS)

    def _prefix(m):
        x = jnp.where(m, 1, 0).astype(I32)
        for k in (1, 2, 4, 8):
            g = _permute(x, jnp.maximum(io - k, 0))
            x = x + jnp.where(io >= k, g, 0)
        return x

    def _process(src_hbm, dst_hbm, table_hbm, out_hbm):
        def _unpack(slot, b):
            for k in range(8):
                v = dst_buf[pl.ds(b * 128 + k * 16, 16)]
                dv[slot, pl.ds(k * 16, 16)] = v & 8191
                pv[slot, pl.ds(k * 16, 16)] = (v >> 13) + s * EPT

        def _val_gather(slot, vslot):
            pltpu.async_copy(src_hbm.at[pv.at[slot]], vbuf.at[vslot], sem_v)

        def chunk_body(p, carry):
            lo = (2 * p + c) * CHUNK
            for k in range(7):
                pltpu.sync_copy(zbuf, acc.at[pl.ds(s * SPAN + k * 56, 56), :])
            pltpu.sync_copy(dst_hbm.at[pl.ds(s * EPT, EPT)],
                            dst_buf.at[pl.ds(0, EPT)])
            plsc.subcore_barrier()

            def scan_body(i, ptr):
                d = dst_buf[pl.ds(i * 16, 16)]
                m = (d >= lo) & (d < lo + CHUNK)
                inc = _prefix(m)
                tgt = ptr + inc - 1
                packed = (d - lo) | ((i * 16 + io) << 13)
                plsc.store_scatter(dst_buf, [tgt], packed, mask=m)
                return ptr + plsc.all_reduce_population_count(m)
            ptr = lax.fori_loop(0, EPT // 16, scan_body,
                                jnp.zeros((16,), I32))
            for k in range(8):
                tgt = ptr + k * 16 + io
                plsc.store_scatter(dst_buf, [tgt], CHUNK + io)
            nb = (ptr[0] + 127) // 128

            @pl.when(nb > 0)
            def _prolog():
                _unpack(0, 0)
                _val_gather(0, 0)

            def batch_body(b, carry2):
                jm2 = b % 2
                jm3 = b % 3
                pltpu.make_async_copy(src_hbm.at[pv.at[jm3]],
                                      vbuf.at[jm2], sem_v).wait()

                @pl.when(b + 1 < nb)
                def _prefetch():
                    _unpack((b + 1) % 3, b + 1)
                    _val_gather((b + 1) % 3, (b + 1) % 2)

                pltpu.async_copy(table_hbm.at[vbuf.at[jm2]],
                                 rows.at[jm2], sem_g).wait()

                @pl.when(b > 0)
                def _drain():
                    pltpu.make_async_copy(rows.at[jm2],
                                          acc.at[dv.at[jm3]], sem_s).wait()
                pltpu.async_copy(rows.at[jm2], acc.at[dv.at[jm3]], sem_s,
                                 add=True)
                return carry2
            lax.fori_loop(0, nb, batch_body, 0)

            @pl.when(nb > 0)
            def _epilog():
                pltpu.make_async_copy(rows.at[0], acc.at[dv.at[0]],
                                      sem_s).wait()
            plsc.subcore_barrier()
            pltpu.sync_copy(acc.at[pl.ds(s * SPAN, SPAN), :],
                            out_hbm.at[pl.ds(lo + s * SPAN, SPAN), :])
            plsc.subcore_barrier()
            return carry
        lax.fori_loop(0, NCHUNK // 2, chunk_body, 0)

    _process(src_hbm0, dst_hbm0, table_hbm0, out_hbm0)


@functools.partial(
    pl.kernel,
    out_type=jax.ShapeDtypeStruct((LP, HW), F32),
    mesh=_mesh,
    scratch_types=[
        pltpu.VMEM((LPT,), I32),
        pltpu.VMEM((LPT,), I32),
        pltpu.VMEM((128, HW), F32),
        pltpu.SemaphoreType.DMA,
    ],
    compiler_params=_sc_params,
)
def _head(i0_hbm, i1_hbm, gm_hbm, gv_hbm, out_hbm, i0_buf, i1_buf, ra, sem):
    c = lax.axis_index("c")
    s = lax.axis_index("s")
    w = s * 2 + c
    base = w * LPT
    pltpu.sync_copy(i0_hbm.at[pl.ds(base, LPT)], i0_buf)
    pltpu.sync_copy(i1_hbm.at[pl.ds(base, LPT)], i1_buf)

    def body(b, carry):
        pltpu.async_copy(gm_hbm.at[i0_buf.at[pl.ds(b * 128, 128)]],
                         ra, sem).wait()
        pltpu.async_copy(gv_hbm.at[i1_buf.at[pl.ds(b * 128, 128)]],
                         ra, sem, add=True).wait()
        pltpu.sync_copy(ra, out_hbm.at[pl.ds(base + b * 128, 128), :])
        return carry
    lax.fori_loop(0, LPT // 128, body, 0)


def _flag_cols(nrows):
    return (lax.broadcasted_iota(I32, (nrows, 16), 1) == 0).astype(F32)


def _proj_body(x_ref, w_ref, b_ref, o_ref):
    h = jnp.dot(x_ref[...], w_ref[...], preferred_element_type=F32) + b_ref[...]
    o_ref[...] = jnp.concatenate([h, _flag_cols(h.shape[0])], axis=1)


_proj = pl.pallas_call(
    _proj_body,
    grid=(NP // SPAN,),
    in_specs=[pl.BlockSpec((SPAN, H), lambda i: (i, 0)),
              pl.BlockSpec((H, H), lambda i: (0, 0)),
              pl.BlockSpec((1, H), lambda i: (0, 0))],
    out_specs=pl.BlockSpec((SPAN, WA), lambda i: (i, 0)),
    out_shape=jax.ShapeDtypeStruct((NP, WA), F32),
)


def _mean_h(agg_ref, xd_ref, wl_ref, bl_ref, wr_ref):
    aggv = agg_ref[:, :H]
    cnt = agg_ref[:, H:H + 1]
    mean = aggv / jnp.maximum(cnt, 1.0)
    h = (jnp.dot(mean, wl_ref[...], preferred_element_type=F32) + bl_ref[...]
         + jnp.dot(xd_ref[:, :H], wr_ref[...], preferred_element_type=F32))
    return jnp.maximum(h, 0.0)


def _comb1_body(agg_ref, xd_ref, wl_ref, bl_ref, wr_ref, o_ref):
    h = _mean_h(agg_ref, xd_ref, wl_ref, bl_ref, wr_ref)
    o_ref[...] = jnp.concatenate([h, _flag_cols(h.shape[0])], axis=1)


_comb1 = pl.pallas_call(
    _comb1_body,
    grid=(NP // SPAN,),
    in_specs=[pl.BlockSpec((SPAN, WA), lambda i: (i, 0)),
              pl.BlockSpec((SPAN, WA), lambda i: (i, 0)),
              pl.BlockSpec((H, H), lambda i: (0, 0)),
              pl.BlockSpec((1, H), lambda i: (0, 0)),
              pl.BlockSpec((H, H), lambda i: (0, 0))],
    out_specs=pl.BlockSpec((SPAN, WA), lambda i: (i, 0)),
    out_shape=jax.ShapeDtypeStruct((NP, WA), F32),
)


def _comb2_body(agg_ref, xd_ref, wl_ref, bl_ref, wr_ref, wc_ref, bc_ref, o_ref):
    h = _mean_h(agg_ref, xd_ref, wl_ref, bl_ref, wr_ref)
    o_ref[...] = jnp.dot(h, wc_ref[...], preferred_element_type=F32) + bc_ref[...]


_comb2 = pl.pallas_call(
    _comb2_body,
    grid=(NP // SPAN,),
    in_specs=[pl.BlockSpec((SPAN, WA), lambda i: (i, 0)),
              pl.BlockSpec((SPAN, WA), lambda i: (i, 0)),
              pl.BlockSpec((H, H), lambda i: (0, 0)),
              pl.BlockSpec((1, H), lambda i: (0, 0)),
              pl.BlockSpec((H, H), lambda i: (0, 0)),
              pl.BlockSpec((H, HW), lambda i: (0, 0)),
              pl.BlockSpec((1, HW), lambda i: (0, 0))],
    out_specs=pl.BlockSpec((SPAN, HW), lambda i: (i, 0)),
    out_shape=jax.ShapeDtypeStruct((NP, HW), F32),
)


def _pad_edges(ei):
    pad = EP - E
    src = jnp.concatenate([ei[0], jnp.zeros((pad,), I32)])
    dst = jnp.concatenate([ei[1], jnp.full((pad,), 1 << 20, I32)])
    return src, dst


def kernel(x_mouse, x_virus, W_mouse, b_mouse, W_virus, b_virus,
           c1mv_Wl, c1mv_bl, c1mv_Wr, c1vm_Wl, c1vm_bl, c1vm_Wr,
           c2mv_Wl, c2mv_bl, c2mv_Wr, c2vm_Wl, c2vm_bl, c2vm_Wr,
           W_cls, b_cls, edge_index_mv, edge_index_vm, edge_label_index):
    src_mv, dst_mv = _pad_edges(edge_index_mv)
    src_vm, dst_vm = _pad_edges(edge_index_vm)

    hm0 = _proj(x_mouse, W_mouse, b_mouse[None])
    hv0 = _proj(x_virus, W_virus, b_virus[None])

    aggv1 = _aggregate(src_mv, dst_mv, hm0)
    aggm1 = _aggregate(src_vm, dst_vm, hv0)
    hv1 = _comb1(aggv1, hv0, c1mv_Wl, c1mv_bl[None], c1mv_Wr)
    hm1 = _comb1(aggm1, hm0, c1vm_Wl, c1vm_bl[None], c1vm_Wr)

    aggv2 = _aggregate(src_mv, dst_mv, hm1)
    aggm2 = _aggregate(src_vm, dst_vm, hv1)

    wc_m = jnp.pad(W_cls[:H], ((0, 0), (0, HW - 2)))
    wc_v = jnp.pad(W_cls[H:], ((0, 0), (0, HW - 2)))
    bc = jnp.pad(b_cls, (0, HW - 2))[None]
    gv = _comb2(aggv2, hv1, c2mv_Wl, c2mv_bl[None], c2mv_Wr, wc_v,
                jnp.zeros((1, HW), F32))
    gm = _comb2(aggm2, hm1, c2vm_Wl, c2vm_bl[None], c2vm_Wr, wc_m, bc)

    i0 = jnp.concatenate([edge_label_index[0], jnp.zeros((LP - L,), I32)])
    i1 = jnp.concatenate([edge_label_index[1], jnp.zeros((LP - L,), I32)])
    out = _head(i0, i1, gm, gv)
    return out[:L, :2]


# speedup vs baseline: 1.5666x; 1.1315x over previous
import functools

import jax
import jax.numpy as jnp
from jax import lax
from jax.experimental import pallas as pl
from jax.experimental.pallas import tpu as pltpu
from jax.experimental.pallas import tpu_sc as plsc

F32 = jnp.float32
I32 = jnp.int32

N = 50000
NP = 50176
H = 128
WA = 144
E = 300000
EPT = 18752
EP = EPT * 16
L = 100000
LP = 102400
HW = 16
CHUNK = 6272
NCHUNK = NP // CHUNK
ACC_ROWS = CHUNK + 16
SPAN = CHUNK // 16
LPT = LP // 32
CAPP = 20480
PK = NCHUNK * 16 * CAPP

_mesh = plsc.VectorSubcoreMesh(
    core_axis_name="c", subcore_axis_name="s", num_cores=2, num_subcores=16)
_sc_params = pltpu.CompilerParams(needs_layout_passes=False,
                                  use_tc_tiling_on_sc=False)


_DNUMS = lax.GatherDimensionNumbers(
    offset_dims=(), collapsed_slice_dims=(0,), start_index_map=(0,))


def _permute(x, idx):
    return lax.gather(x, idx[:, None], _DNUMS, slice_sizes=(1,),
                      mode=lax.GatherScatterMode.PROMISE_IN_BOUNDS)


def _prefix(m):
    io = lax.iota(I32, 16)
    x = jnp.where(m, 1, 0).astype(I32)
    for k in (1, 2, 4, 8):
        g = _permute(x, jnp.maximum(io - k, 0))
        x = x + jnp.where(io >= k, g, 0)
    return x


@functools.partial(
    pl.kernel,
    out_type=[jax.ShapeDtypeStruct((PK,), I32),
              jax.ShapeDtypeStruct((PK,), I32)],
    mesh=_mesh,
    scratch_types=[
        pltpu.VMEM((EPT,), I32),
        pltpu.VMEM((CAPP,), I32),
    ],
    compiler_params=_sc_params,
)
def _partition(dst_hbm0, dst_hbm1, out_hbm0, out_hbm1, dst_sbuf, cbuf):
    c = lax.axis_index("c")
    s = lax.axis_index("s")
    io = lax.iota(I32, 16)

    for dst_hbm, out_hbm in ((dst_hbm0, out_hbm0), (dst_hbm1, out_hbm1)):
        pltpu.sync_copy(dst_hbm.at[pl.ds(s * EPT, EPT)], dst_sbuf)

        def chunk_body(p, carry):
            chunk = 2 * p + c
            lo = chunk * CHUNK

            def scan_body(i, ptr):
                d = dst_sbuf[pl.ds(i * 16, 16)]
                m = (d >= lo) & (d < lo + CHUNK)
                inc = _prefix(m)
                tgt = ptr + inc - 1 + 128
                packed = (d - lo) | ((i * 16 + io) << 13)
                plsc.store_scatter(cbuf, [tgt], packed, mask=m)
                return ptr + plsc.all_reduce_population_count(m)
            ptr = lax.fori_loop(0, EPT // 16, scan_body,
                                jnp.zeros((16,), I32))
            for k in range(8):
                tgt = ptr + k * 16 + io + 128
                plsc.store_scatter(cbuf, [tgt], CHUNK + io)
            nbv = (ptr + 127) >> 7
            cbuf[pl.ds(0, 16)] = nbv
            base = (chunk * 16 + s) * CAPP
            n2 = (nbv[0] * 128 + 128 + 2047) // 2048

            def wr(j, cc):
                pltpu.sync_copy(cbuf.at[pl.ds(j * 2048, 2048)],
                                out_hbm.at[pl.ds(base + j * 2048, 2048)])
                return cc
            lax.fori_loop(0, n2, wr, 0)
            return carry
        lax.fori_loop(0, NCHUNK // 2, chunk_body, 0)

@functools.partial(
    pl.kernel,
    out_type=jax.ShapeDtypeStruct((NP, WA), F32),
    mesh=_mesh,
    scratch_types=[
        pltpu.VMEM((CAPP,), I32),
        pltpu.VMEM((2, 128, WA), F32),
        pltpu.VMEM((2, 128), I32),
        pltpu.VMEM((3, 128), I32),
        pltpu.VMEM((3, 128), I32),
        pltpu.VMEM((56, WA), F32),
        pltpu.VMEM_SHARED((ACC_ROWS, WA), F32),
        pltpu.SemaphoreType.DMA,
        pltpu.SemaphoreType.DMA,
        pltpu.SemaphoreType.DMA,
    ],
    compiler_params=_sc_params,
)
def _aggregate(src_hbm0, packed_hbm0, table_hbm0, out_hbm0,
               dst_buf, rows, vbuf, pv, dv, zbuf, acc, sem_v, sem_g, sem_s):
    c = lax.axis_index("c")
    s = lax.axis_index("s")
    zvec = jnp.zeros((16,), F32)

    def _zb(i, carry):
        for k in range(9):
            zbuf[i, pl.ds(k * 16, 16)] = zvec
        return carry
    lax.fori_loop(0, 56, _zb, 0)

    def _process(src_hbm, packed_hbm, table_hbm, out_hbm):
        def _unpack(slot, b):
            for k in range(8):
                v = dst_buf[pl.ds(128 + b * 128 + k * 16, 16)]
                dv[slot, pl.ds(k * 16, 16)] = v & 8191
                pv[slot, pl.ds(k * 16, 16)] = (v >> 13) + s * EPT

        def _val_gather(slot, vslot):
            pltpu.async_copy(src_hbm.at[pv.at[slot]], vbuf.at[vslot], sem_v)

        def chunk_body(p, carry):
            chunk = 2 * p + c
            lo = chunk * CHUNK
            for k in range(7):
                pltpu.sync_copy(zbuf, acc.at[pl.ds(s * SPAN + k * 56, 56), :])
            base = (chunk * 16 + s) * CAPP
            pltpu.sync_copy(packed_hbm.at[pl.ds(base, 16)],
                            dst_buf.at[pl.ds(0, 16)])
            nb = dst_buf[pl.ds(0, 16)][0]
            n2 = (nb * 128 + 128 + 2047) // 2048

            def stg(j, cc):
                pltpu.sync_copy(packed_hbm.at[pl.ds(base + j * 2048, 2048)],
                                dst_buf.at[pl.ds(j * 2048, 2048)])
                return cc
            lax.fori_loop(0, n2, stg, 0)
            plsc.subcore_barrier()

            @pl.when(nb > 0)
            def _prolog():
                _unpack(0, 0)
                _val_gather(0, 0)

            def batch_body(b, carry2):
                jm2 = b % 2
                jm3 = b % 3
                pltpu.make_async_copy(src_hbm.at[pv.at[jm3]],
                                      vbuf.at[jm2], sem_v).wait()

                @pl.when(b + 1 < nb)
                def _prefetch():
                    _unpack((b + 1) % 3, b + 1)
                    _val_gather((b + 1) % 3, (b + 1) % 2)

                pltpu.async_copy(table_hbm.at[vbuf.at[jm2]],
                                 rows.at[jm2], sem_g).wait()

                @pl.when(b > 0)
                def _drain():
                    pltpu.make_async_copy(rows.at[jm2],
                                          acc.at[dv.at[jm3]], sem_s).wait()
                pltpu.async_copy(rows.at[jm2], acc.at[dv.at[jm3]], sem_s,
                                 add=True)
                return carry2
            lax.fori_loop(0, nb, batch_body, 0)

            @pl.when(nb > 0)
            def _epilog():
                pltpu.make_async_copy(rows.at[0], acc.at[dv.at[0]],
                                      sem_s).wait()
            plsc.subcore_barrier()
            pltpu.sync_copy(acc.at[pl.ds(s * SPAN, SPAN), :],
                            out_hbm.at[pl.ds(lo + s * SPAN, SPAN), :])
            plsc.subcore_barrier()
            return carry
        lax.fori_loop(0, NCHUNK // 2, chunk_body, 0)

    _process(src_hbm0, packed_hbm0, table_hbm0, out_hbm0)


@functools.partial(
    pl.kernel,
    out_type=jax.ShapeDtypeStruct((LP, HW), F32),
    mesh=_mesh,
    scratch_types=[
        pltpu.VMEM((LPT,), I32),
        pltpu.VMEM((LPT,), I32),
        pltpu.VMEM((128, HW), F32),
        pltpu.SemaphoreType.DMA,
    ],
    compiler_params=_sc_params,
)
def _head(i0_hbm, i1_hbm, gm_hbm, gv_hbm, out_hbm, i0_buf, i1_buf, ra, sem):
    c = lax.axis_index("c")
    s = lax.axis_index("s")
    w = s * 2 + c
    base = w * LPT
    pltpu.sync_copy(i0_hbm.at[pl.ds(base, LPT)], i0_buf)
    pltpu.sync_copy(i1_hbm.at[pl.ds(base, LPT)], i1_buf)

    def body(b, carry):
        pltpu.async_copy(gm_hbm.at[i0_buf.at[pl.ds(b * 128, 128)]],
                         ra, sem).wait()
        pltpu.async_copy(gv_hbm.at[i1_buf.at[pl.ds(b * 128, 128)]],
                         ra, sem, add=True).wait()
        pltpu.sync_copy(ra, out_hbm.at[pl.ds(base + b * 128, 128), :])
        return carry
    lax.fori_loop(0, LPT // 128, body, 0)


def _flag_cols(nrows):
    return (lax.broadcasted_iota(I32, (nrows, 16), 1) == 0).astype(F32)


def _proj_body(x_ref, w_ref, b_ref, o_ref):
    h = jnp.dot(x_ref[...], w_ref[...], preferred_element_type=F32) + b_ref[...]
    o_ref[...] = jnp.concatenate([h, _flag_cols(h.shape[0])], axis=1)


_proj = pl.pallas_call(
    _proj_body,
    grid=(NP // SPAN,),
    in_specs=[pl.BlockSpec((SPAN, H), lambda i: (i, 0)),
              pl.BlockSpec((H, H), lambda i: (0, 0)),
              pl.BlockSpec((1, H), lambda i: (0, 0))],
    out_specs=pl.BlockSpec((SPAN, WA), lambda i: (i, 0)),
    out_shape=jax.ShapeDtypeStruct((NP, WA), F32),
)


def _mean_h(agg_ref, xd_ref, wl_ref, bl_ref, wr_ref):
    aggv = agg_ref[:, :H]
    cnt = agg_ref[:, H:H + 1]
    mean = aggv / jnp.maximum(cnt, 1.0)
    h = (jnp.dot(mean, wl_ref[...], preferred_element_type=F32) + bl_ref[...]
         + jnp.dot(xd_ref[:, :H], wr_ref[...], preferred_element_type=F32))
    return jnp.maximum(h, 0.0)


def _comb1_body(agg_ref, xd_ref, wl_ref, bl_ref, wr_ref, o_ref):
    h = _mean_h(agg_ref, xd_ref, wl_ref, bl_ref, wr_ref)
    o_ref[...] = jnp.concatenate([h, _flag_cols(h.shape[0])], axis=1)


_comb1 = pl.pallas_call(
    _comb1_body,
    grid=(NP // SPAN,),
    in_specs=[pl.BlockSpec((SPAN, WA), lambda i: (i, 0)),
              pl.BlockSpec((SPAN, WA), lambda i: (i, 0)),
              pl.BlockSpec((H, H), lambda i: (0, 0)),
              pl.BlockSpec((1, H), lambda i: (0, 0)),
              pl.BlockSpec((H, H), lambda i: (0, 0))],
    out_specs=pl.BlockSpec((SPAN, WA), lambda i: (i, 0)),
    out_shape=jax.ShapeDtypeStruct((NP, WA), F32),
)


def _comb2_body(agg_ref, xd_ref, wl_ref, bl_ref, wr_ref, wc_ref, bc_ref, o_ref):
    h = _mean_h(agg_ref, xd_ref, wl_ref, bl_ref, wr_ref)
    o_ref[...] = jnp.dot(h, wc_ref[...], preferred_element_type=F32) + bc_ref[...]


_comb2 = pl.pallas_call(
    _comb2_body,
    grid=(NP // SPAN,),
    in_specs=[pl.BlockSpec((SPAN, WA), lambda i: (i, 0)),
              pl.BlockSpec((SPAN, WA), lambda i: (i, 0)),
              pl.BlockSpec((H, H), lambda i: (0, 0)),
              pl.BlockSpec((1, H), lambda i: (0, 0)),
              pl.BlockSpec((H, H), lambda i: (0, 0)),
              pl.BlockSpec((H, HW), lambda i: (0, 0)),
              pl.BlockSpec((1, HW), lambda i: (0, 0))],
    out_specs=pl.BlockSpec((SPAN, HW), lambda i: (i, 0)),
    out_shape=jax.ShapeDtypeStruct((NP, HW), F32),
)


def _pad_edges(ei):
    pad = EP - E
    src = jnp.concatenate([ei[0], jnp.zeros((pad,), I32)])
    dst = jnp.concatenate([ei[1], jnp.full((pad,), 1 << 20, I32)])
    return src, dst


def kernel(x_mouse, x_virus, W_mouse, b_mouse, W_virus, b_virus,
           c1mv_Wl, c1mv_bl, c1mv_Wr, c1vm_Wl, c1vm_bl, c1vm_Wr,
           c2mv_Wl, c2mv_bl, c2mv_Wr, c2vm_Wl, c2vm_bl, c2vm_Wr,
           W_cls, b_cls, edge_index_mv, edge_index_vm, edge_label_index):
    src_mv, dst_mv = _pad_edges(edge_index_mv)
    src_vm, dst_vm = _pad_edges(edge_index_vm)

    pk_mv, pk_vm = _partition(dst_mv, dst_vm)
    hm0 = _proj(x_mouse, W_mouse, b_mouse[None])
    hv0 = _proj(x_virus, W_virus, b_virus[None])

    aggv1 = _aggregate(src_mv, pk_mv, hm0)
    aggm1 = _aggregate(src_vm, pk_vm, hv0)
    hv1 = _comb1(aggv1, hv0, c1mv_Wl, c1mv_bl[None], c1mv_Wr)
    hm1 = _comb1(aggm1, hm0, c1vm_Wl, c1vm_bl[None], c1vm_Wr)

    aggv2 = _aggregate(src_mv, pk_mv, hm1)
    aggm2 = _aggregate(src_vm, pk_vm, hv1)

    wc_m = jnp.pad(W_cls[:H], ((0, 0), (0, HW - 2)))
    wc_v = jnp.pad(W_cls[H:], ((0, 0), (0, HW - 2)))
    bc = jnp.pad(b_cls, (0, HW - 2))[None]
    gv = _comb2(aggv2, hv1, c2mv_Wl, c2mv_bl[None], c2mv_Wr, wc_v,
                jnp.zeros((1, HW), F32))
    gm = _comb2(aggm2, hm1, c2vm_Wl, c2vm_bl[None], c2vm_Wr, wc_m, bc)

    i0 = jnp.concatenate([edge_label_index[0], jnp.zeros((LP - L,), I32)])
    i1 = jnp.concatenate([edge_label_index[1], jnp.zeros((LP - L,), I32)])
    out = _head(i0, i1, gm, gv)
    return out[:L, :2]


# speedup vs baseline: 1.5824x; 1.0101x over previous
import functools

import jax
import jax.numpy as jnp
from jax import lax
from jax.experimental import pallas as pl
from jax.experimental.pallas import tpu as pltpu
from jax.experimental.pallas import tpu_sc as plsc

F32 = jnp.float32
I32 = jnp.int32

N = 50000
NP = 50176
H = 128
WA = 144
E = 300000
EPT = 18752
EP = EPT * 16
L = 100000
LP = 102400
HW = 16
CHUNK = 6272
NCHUNK = NP // CHUNK
ACC_ROWS = CHUNK + 16
SPAN = CHUNK // 16
LPT = LP // 32
CAPP = 20480
PK = NCHUNK * 16 * CAPP

_mesh = plsc.VectorSubcoreMesh(
    core_axis_name="c", subcore_axis_name="s", num_cores=2, num_subcores=16)
_sc_params = pltpu.CompilerParams(needs_layout_passes=False,
                                  use_tc_tiling_on_sc=False)


_DNUMS = lax.GatherDimensionNumbers(
    offset_dims=(), collapsed_slice_dims=(0,), start_index_map=(0,))


def _permute(x, idx):
    return lax.gather(x, idx[:, None], _DNUMS, slice_sizes=(1,),
                      mode=lax.GatherScatterMode.PROMISE_IN_BOUNDS)


def _prefix(m):
    io = lax.iota(I32, 16)
    x = jnp.where(m, 1, 0).astype(I32)
    for k in (1, 2, 4, 8):
        g = _permute(x, jnp.maximum(io - k, 0))
        x = x + jnp.where(io >= k, g, 0)
    return x


@functools.partial(
    pl.kernel,
    out_type=[jax.ShapeDtypeStruct((PK,), I32),
              jax.ShapeDtypeStruct((PK,), I32)],
    mesh=_mesh,
    scratch_types=[
        pltpu.VMEM((EPT,), I32),
        pltpu.VMEM((CAPP,), I32),
    ],
    compiler_params=_sc_params,
)
def _partition(dst_hbm0, dst_hbm1, out_hbm0, out_hbm1, dst_sbuf, cbuf):
    c = lax.axis_index("c")
    s = lax.axis_index("s")
    io = lax.iota(I32, 16)

    for dst_hbm, out_hbm in ((dst_hbm0, out_hbm0), (dst_hbm1, out_hbm1)):
        pltpu.sync_copy(dst_hbm.at[pl.ds(s * EPT, EPT)], dst_sbuf)

        def chunk_body(p, carry):
            chunk = 2 * p + c
            lo = chunk * CHUNK

            def scan_body(i, ptr):
                d = dst_sbuf[pl.ds(i * 16, 16)]
                m = (d >= lo) & (d < lo + CHUNK)
                inc = _prefix(m)
                tgt = ptr + inc - 1 + 128
                packed = (d - lo) | ((i * 16 + io) << 13)
                plsc.store_scatter(cbuf, [tgt], packed, mask=m)
                return ptr + plsc.all_reduce_population_count(m)
            ptr = lax.fori_loop(0, EPT // 16, scan_body,
                                jnp.zeros((16,), I32))
            for k in range(8):
                tgt = ptr + k * 16 + io + 128
                plsc.store_scatter(cbuf, [tgt], CHUNK + io)
            nbv = (ptr + 127) >> 7
            cbuf[pl.ds(0, 16)] = nbv
            base = (chunk * 16 + s) * CAPP
            n2 = (nbv[0] * 128 + 128 + 2047) // 2048

            def wr(j, cc):
                pltpu.sync_copy(cbuf.at[pl.ds(j * 2048, 2048)],
                                out_hbm.at[pl.ds(base + j * 2048, 2048)])
                return cc
            lax.fori_loop(0, n2, wr, 0)
            return carry
        lax.fori_loop(0, NCHUNK // 2, chunk_body, 0)

@functools.partial(
    pl.kernel,
    out_type=jax.ShapeDtypeStruct((NP, WA), F32),
    mesh=_mesh,
    scratch_types=[
        pltpu.VMEM((CAPP,), I32),
        pltpu.VMEM((2, 128, WA), F32),
        pltpu.VMEM((3, 128), I32),
        pltpu.VMEM((4, 128), I32),
        pltpu.VMEM((4, 128), I32),
        pltpu.VMEM((56, WA), F32),
        pltpu.VMEM_SHARED((ACC_ROWS, WA), F32),
        pltpu.SemaphoreType.DMA,
        pltpu.SemaphoreType.DMA,
        pltpu.SemaphoreType.DMA,
    ],
    compiler_params=_sc_params,
)
def _aggregate(src_hbm0, packed_hbm0, table_hbm0, out_hbm0,
               dst_buf, rows, vbuf, pv, dv, zbuf, acc, sem_v, sem_g, sem_s):
    c = lax.axis_index("c")
    s = lax.axis_index("s")
    zvec = jnp.zeros((16,), F32)

    def _zb(i, carry):
        for k in range(9):
            zbuf[i, pl.ds(k * 16, 16)] = zvec
        return carry
    lax.fori_loop(0, 56, _zb, 0)

    def _process(src_hbm, packed_hbm, table_hbm, out_hbm):
        def _unpack(slot, b):
            for k in range(8):
                v = dst_buf[pl.ds(128 + b * 128 + k * 16, 16)]
                dv[slot, pl.ds(k * 16, 16)] = v & 8191
                pv[slot, pl.ds(k * 16, 16)] = (v >> 13) + s * EPT

        def _val_gather(slot, vslot):
            pltpu.async_copy(src_hbm.at[pv.at[slot]], vbuf.at[vslot], sem_v)

        def chunk_body(p, carry):
            chunk = 2 * p + c
            lo = chunk * CHUNK
            for k in range(7):
                pltpu.sync_copy(zbuf, acc.at[pl.ds(s * SPAN + k * 56, 56), :])
            base = (chunk * 16 + s) * CAPP
            pltpu.sync_copy(packed_hbm.at[pl.ds(base, 16)],
                            dst_buf.at[pl.ds(0, 16)])
            nb = dst_buf[pl.ds(0, 16)][0]
            n2 = (nb * 128 + 128 + 2047) // 2048

            def stg(j, cc):
                pltpu.sync_copy(packed_hbm.at[pl.ds(base + j * 2048, 2048)],
                                dst_buf.at[pl.ds(j * 2048, 2048)])
                return cc
            lax.fori_loop(0, n2, stg, 0)
            plsc.subcore_barrier()

            @pl.when(nb > 0)
            def _prolog():
                _unpack(0, 0)
                _val_gather(0, 0)

            @pl.when(nb > 1)
            def _prolog2():
                _unpack(1, 1)
                _val_gather(1, 1)

            @pl.when(nb > 0)
            def _prolog3():
                pltpu.make_async_copy(src_hbm.at[pv.at[0]],
                                      vbuf.at[0], sem_v).wait()
                pltpu.async_copy(table_hbm.at[vbuf.at[0]], rows.at[0], sem_g)

            def batch_body(b, carry2):
                @pl.when(b + 2 < nb)
                def _prefetch():
                    _unpack((b + 2) % 4, b + 2)
                    _val_gather((b + 2) % 4, (b + 2) % 3)

                @pl.when(b > 0)
                def _drain():
                    pltpu.make_async_copy(rows.at[b % 2],
                                          acc.at[dv.at[0]], sem_s).wait()

                @pl.when(b + 1 < nb)
                def _next_row():
                    pltpu.make_async_copy(src_hbm.at[pv.at[(b + 1) % 4]],
                                          vbuf.at[(b + 1) % 3], sem_v).wait()
                    pltpu.async_copy(table_hbm.at[vbuf.at[(b + 1) % 3]],
                                     rows.at[(b + 1) % 2], sem_g)

                pltpu.make_async_copy(table_hbm.at[vbuf.at[b % 3]],
                                      rows.at[b % 2], sem_g).wait()
                pltpu.async_copy(rows.at[b % 2], acc.at[dv.at[b % 4]], sem_s,
                                 add=True)
                return carry2
            lax.fori_loop(0, nb, batch_body, 0)

            @pl.when(nb > 0)
            def _epilog():
                pltpu.make_async_copy(rows.at[0], acc.at[dv.at[0]],
                                      sem_s).wait()
            plsc.subcore_barrier()
            pltpu.sync_copy(acc.at[pl.ds(s * SPAN, SPAN), :],
                            out_hbm.at[pl.ds(lo + s * SPAN, SPAN), :])
            plsc.subcore_barrier()
            return carry
        lax.fori_loop(0, NCHUNK // 2, chunk_body, 0)

    _process(src_hbm0, packed_hbm0, table_hbm0, out_hbm0)


@functools.partial(
    pl.kernel,
    out_type=jax.ShapeDtypeStruct((LP, HW), F32),
    mesh=_mesh,
    scratch_types=[
        pltpu.VMEM((LPT,), I32),
        pltpu.VMEM((LPT,), I32),
        pltpu.VMEM((128, HW), F32),
        pltpu.SemaphoreType.DMA,
    ],
    compiler_params=_sc_params,
)
def _head(i0_hbm, i1_hbm, gm_hbm, gv_hbm, out_hbm, i0_buf, i1_buf, ra, sem):
    c = lax.axis_index("c")
    s = lax.axis_index("s")
    w = s * 2 + c
    base = w * LPT
    pltpu.sync_copy(i0_hbm.at[pl.ds(base, LPT)], i0_buf)
    pltpu.sync_copy(i1_hbm.at[pl.ds(base, LPT)], i1_buf)

    def body(b, carry):
        pltpu.async_copy(gm_hbm.at[i0_buf.at[pl.ds(b * 128, 128)]],
                         ra, sem).wait()
        pltpu.async_copy(gv_hbm.at[i1_buf.at[pl.ds(b * 128, 128)]],
                         ra, sem, add=True).wait()
        pltpu.sync_copy(ra, out_hbm.at[pl.ds(base + b * 128, 128), :])
        return carry
    lax.fori_loop(0, LPT // 128, body, 0)


def _flag_cols(nrows):
    return (lax.broadcasted_iota(I32, (nrows, 16), 1) == 0).astype(F32)


def _proj_body(x_ref, w_ref, b_ref, o_ref):
    h = jnp.dot(x_ref[...], w_ref[...], preferred_element_type=F32) + b_ref[...]
    o_ref[...] = jnp.concatenate([h, _flag_cols(h.shape[0])], axis=1)


_proj = pl.pallas_call(
    _proj_body,
    grid=(NP // SPAN,),
    in_specs=[pl.BlockSpec((SPAN, H), lambda i: (i, 0)),
              pl.BlockSpec((H, H), lambda i: (0, 0)),
              pl.BlockSpec((1, H), lambda i: (0, 0))],
    out_specs=pl.BlockSpec((SPAN, WA), lambda i: (i, 0)),
    out_shape=jax.ShapeDtypeStruct((NP, WA), F32),
)


def _mean_h(agg_ref, xd_ref, wl_ref, bl_ref, wr_ref):
    aggv = agg_ref[:, :H]
    cnt = agg_ref[:, H:H + 1]
    mean = aggv / jnp.maximum(cnt, 1.0)
    h = (jnp.dot(mean, wl_ref[...], preferred_element_type=F32) + bl_ref[...]
         + jnp.dot(xd_ref[:, :H], wr_ref[...], preferred_element_type=F32))
    return jnp.maximum(h, 0.0)


def _comb1_body(agg_ref, xd_ref, wl_ref, bl_ref, wr_ref, o_ref):
    h = _mean_h(agg_ref, xd_ref, wl_ref, bl_ref, wr_ref)
    o_ref[...] = jnp.concatenate([h, _flag_cols(h.shape[0])], axis=1)


_comb1 = pl.pallas_call(
    _comb1_body,
    grid=(NP // SPAN,),
    in_specs=[pl.BlockSpec((SPAN, WA), lambda i: (i, 0)),
              pl.BlockSpec((SPAN, WA), lambda i: (i, 0)),
              pl.BlockSpec((H, H), lambda i: (0, 0)),
              pl.BlockSpec((1, H), lambda i: (0, 0)),
              pl.BlockSpec((H, H), lambda i: (0, 0))],
    out_specs=pl.BlockSpec((SPAN, WA), lambda i: (i, 0)),
    out_shape=jax.ShapeDtypeStruct((NP, WA), F32),
)


def _comb2_body(agg_ref, xd_ref, wl_ref, bl_ref, wr_ref, wc_ref, bc_ref, o_ref):
    h = _mean_h(agg_ref, xd_ref, wl_ref, bl_ref, wr_ref)
    o_ref[...] = jnp.dot(h, wc_ref[...], preferred_element_type=F32) + bc_ref[...]


_comb2 = pl.pallas_call(
    _comb2_body,
    grid=(NP // SPAN,),
    in_specs=[pl.BlockSpec((SPAN, WA), lambda i: (i, 0)),
              pl.BlockSpec((SPAN, WA), lambda i: (i, 0)),
              pl.BlockSpec((H, H), lambda i: (0, 0)),
              pl.BlockSpec((1, H), lambda i: (0, 0)),
              pl.BlockSpec((H, H), lambda i: (0, 0)),
              pl.BlockSpec((H, HW), lambda i: (0, 0)),
              pl.BlockSpec((1, HW), lambda i: (0, 0))],
    out_specs=pl.BlockSpec((SPAN, HW), lambda i: (i, 0)),
    out_shape=jax.ShapeDtypeStruct((NP, HW), F32),
)


def _pad_edges(ei):
    pad = EP - E
    src = jnp.concatenate([ei[0], jnp.zeros((pad,), I32)])
    dst = jnp.concatenate([ei[1], jnp.full((pad,), 1 << 20, I32)])
    return src, dst


def kernel(x_mouse, x_virus, W_mouse, b_mouse, W_virus, b_virus,
           c1mv_Wl, c1mv_bl, c1mv_Wr, c1vm_Wl, c1vm_bl, c1vm_Wr,
           c2mv_Wl, c2mv_bl, c2mv_Wr, c2vm_Wl, c2vm_bl, c2vm_Wr,
           W_cls, b_cls, edge_index_mv, edge_index_vm, edge_label_index):
    src_mv, dst_mv = _pad_edges(edge_index_mv)
    src_vm, dst_vm = _pad_edges(edge_index_vm)

    pk_mv, pk_vm = _partition(dst_mv, dst_vm)
    hm0 = _proj(x_mouse, W_mouse, b_mouse[None])
    hv0 = _proj(x_virus, W_virus, b_virus[None])

    aggv1 = _aggregate(src_mv, pk_mv, hm0)
    aggm1 = _aggregate(src_vm, pk_vm, hv0)
    hv1 = _comb1(aggv1, hv0, c1mv_Wl, c1mv_bl[None], c1mv_Wr)
    hm1 = _comb1(aggm1, hm0, c1vm_Wl, c1vm_bl[None], c1vm_Wr)

    aggv2 = _aggregate(src_mv, pk_mv, hm1)
    aggm2 = _aggregate(src_vm, pk_vm, hv1)

    wc_m = jnp.pad(W_cls[:H], ((0, 0), (0, HW - 2)))
    wc_v = jnp.pad(W_cls[H:], ((0, 0), (0, HW - 2)))
    bc = jnp.pad(b_cls, (0, HW - 2))[None]
    gv = _comb2(aggv2, hv1, c2mv_Wl, c2mv_bl[None], c2mv_Wr, wc_v,
                jnp.zeros((1, HW), F32))
    gm = _comb2(aggm2, hm1, c2vm_Wl, c2vm_bl[None], c2vm_Wr, wc_m, bc)

    i0 = jnp.concatenate([edge_label_index[0], jnp.zeros((LP - L,), I32)])
    i1 = jnp.concatenate([edge_label_index[1], jnp.zeros((LP - L,), I32)])
    out = _head(i0, i1, gm, gv)
    return out[:L, :2]


# speedup vs baseline: 1.5832x; 1.0005x over previous
import functools

import jax
import jax.numpy as jnp
from jax import lax
from jax.experimental import pallas as pl
from jax.experimental.pallas import tpu as pltpu
from jax.experimental.pallas import tpu_sc as plsc

F32 = jnp.float32
I32 = jnp.int32

N = 50000
NP = 50176
H = 128
WA = 144
E = 300000
EPT = 18752
EP = EPT * 16
L = 100000
LP = 102400
HW = 16
CHUNK = 6272
NCHUNK = NP // CHUNK
ACC_ROWS = CHUNK + 16
SPAN = CHUNK // 16
LPT = LP // 32
CAPP = 20480
PK = NCHUNK * 16 * CAPP

_mesh = plsc.VectorSubcoreMesh(
    core_axis_name="c", subcore_axis_name="s", num_cores=2, num_subcores=16)
_sc_params = pltpu.CompilerParams(needs_layout_passes=False,
                                  use_tc_tiling_on_sc=False)


_DNUMS = lax.GatherDimensionNumbers(
    offset_dims=(), collapsed_slice_dims=(0,), start_index_map=(0,))


def _permute(x, idx):
    return lax.gather(x, idx[:, None], _DNUMS, slice_sizes=(1,),
                      mode=lax.GatherScatterMode.PROMISE_IN_BOUNDS)


def _prefix(m):
    io = lax.iota(I32, 16)
    x = jnp.where(m, 1, 0).astype(I32)
    for k in (1, 2, 4, 8):
        g = _permute(x, jnp.maximum(io - k, 0))
        x = x + jnp.where(io >= k, g, 0)
    return x


@functools.partial(
    pl.kernel,
    out_type=[jax.ShapeDtypeStruct((PK,), I32),
              jax.ShapeDtypeStruct((PK,), I32)],
    mesh=_mesh,
    scratch_types=[
        pltpu.VMEM((EPT,), I32),
        pltpu.VMEM((CAPP,), I32),
    ],
    compiler_params=_sc_params,
)
def _partition(dst_hbm0, dst_hbm1, out_hbm0, out_hbm1, dst_sbuf, cbuf):
    c = lax.axis_index("c")
    s = lax.axis_index("s")
    io = lax.iota(I32, 16)

    for dst_hbm, out_hbm in ((dst_hbm0, out_hbm0), (dst_hbm1, out_hbm1)):
        pltpu.sync_copy(dst_hbm.at[pl.ds(s * EPT, EPT)], dst_sbuf)

        def chunk_body(p, carry):
            chunk = 2 * p + c
            lo = chunk * CHUNK

            def scan_body(i, ptr):
                d = dst_sbuf[pl.ds(i * 16, 16)]
                m = (d >= lo) & (d < lo + CHUNK)
                inc = _prefix(m)
                tgt = ptr + inc - 1 + 128
                packed = (d - lo) | ((i * 16 + io) << 13)
                plsc.store_scatter(cbuf, [tgt], packed, mask=m)
                return ptr + plsc.all_reduce_population_count(m)
            ptr = lax.fori_loop(0, EPT // 16, scan_body,
                                jnp.zeros((16,), I32))
            for k in range(8):
                tgt = ptr + k * 16 + io + 128
                plsc.store_scatter(cbuf, [tgt], CHUNK + io)
            nbv = (ptr + 127) >> 7
            cbuf[pl.ds(0, 16)] = nbv
            base = (chunk * 16 + s) * CAPP
            n2 = (nbv[0] * 128 + 128 + 2047) // 2048

            def wr(j, cc):
                pltpu.sync_copy(cbuf.at[pl.ds(j * 2048, 2048)],
                                out_hbm.at[pl.ds(base + j * 2048, 2048)])
                return cc
            lax.fori_loop(0, n2, wr, 0)
            return carry
        lax.fori_loop(0, NCHUNK // 2, chunk_body, 0)

@functools.partial(
    pl.kernel,
    out_type=jax.ShapeDtypeStruct((NP, WA), F32),
    mesh=_mesh,
    scratch_types=[
        pltpu.VMEM((CAPP,), I32),
        pltpu.VMEM((2, 128, WA), F32),
        pltpu.VMEM((3, 128), I32),
        pltpu.VMEM((4, 128), I32),
        pltpu.VMEM((4, 128), I32),
        pltpu.VMEM((56, WA), F32),
        pltpu.VMEM_SHARED((ACC_ROWS, WA), F32),
        pltpu.SemaphoreType.DMA,
        pltpu.SemaphoreType.DMA,
        pltpu.SemaphoreType.DMA,
    ],
    compiler_params=_sc_params,
)
def _aggregate(src_hbm0, packed_hbm0, table_hbm0, out_hbm0,
               dst_buf, rows, vbuf, pv, dv, zbuf, acc, sem_v, sem_g, sem_s):
    c = lax.axis_index("c")
    s = lax.axis_index("s")
    zvec = jnp.zeros((16,), F32)

    def _zb(i, carry):
        for k in range(9):
            zbuf[i, pl.ds(k * 16, 16)] = zvec
        return carry
    lax.fori_loop(0, 56, _zb, 0)

    def _process(src_hbm, packed_hbm, table_hbm, out_hbm):
        def _unpack(slot, b):
            for k in range(8):
                v = dst_buf[pl.ds(128 + b * 128 + k * 16, 16)]
                dv[slot, pl.ds(k * 16, 16)] = v & 8191
                pv[slot, pl.ds(k * 16, 16)] = (v >> 13) + s * EPT

        def _val_gather(slot, vslot):
            pltpu.async_copy(src_hbm.at[pv.at[slot]], vbuf.at[vslot], sem_v)

        def chunk_body(p, carry):
            chunk = 2 * p + c
            lo = chunk * CHUNK
            for k in range(7):
                pltpu.sync_copy(zbuf, acc.at[pl.ds(s * SPAN + k * 56, 56), :])
            base = (chunk * 16 + s) * CAPP
            pltpu.sync_copy(packed_hbm.at[pl.ds(base, 16)],
                            dst_buf.at[pl.ds(0, 16)])
            nb = dst_buf[pl.ds(0, 16)][0]
            n2 = (nb * 128 + 128 + 2047) // 2048

            def stg(j, cc):
                pltpu.sync_copy(packed_hbm.at[pl.ds(base + j * 2048, 2048)],
                                dst_buf.at[pl.ds(j * 2048, 2048)])
                return cc
            lax.fori_loop(0, n2, stg, 0)
            plsc.subcore_barrier()

            @pl.when(nb > 0)
            def _prolog():
                _unpack(0, 0)
                _val_gather(0, 0)

            @pl.when(nb > 1)
            def _prolog2():
                _unpack(1, 1)
                _val_gather(1, 1)

            @pl.when(nb > 0)
            def _prolog3():
                pltpu.make_async_copy(src_hbm.at[pv.at[0]],
                                      vbuf.at[0], sem_v).wait()
                pltpu.async_copy(table_hbm.at[vbuf.at[0]], rows.at[0], sem_g)

            def batch_body(b, carry2):
                @pl.when(b + 2 < nb)
                def _prefetch():
                    _unpack((b + 2) % 4, b + 2)
                    _val_gather((b + 2) % 4, (b + 2) % 3)

                @pl.when(b > 0)
                def _drain():
                    pltpu.make_async_copy(rows.at[b % 2],
                                          acc.at[dv.at[0]], sem_s).wait()

                @pl.when(b + 1 < nb)
                def _next_row():
                    pltpu.make_async_copy(src_hbm.at[pv.at[(b + 1) % 4]],
                                          vbuf.at[(b + 1) % 3], sem_v).wait()
                    pltpu.async_copy(table_hbm.at[vbuf.at[(b + 1) % 3]],
                                     rows.at[(b + 1) % 2], sem_g)

                pltpu.make_async_copy(table_hbm.at[vbuf.at[b % 3]],
                                      rows.at[b % 2], sem_g).wait()
                pltpu.async_copy(rows.at[b % 2], acc.at[dv.at[b % 4]], sem_s,
                                 add=True)
                return carry2
            lax.fori_loop(0, nb, batch_body, 0)

            @pl.when(nb > 0)
            def _epilog():
                pltpu.make_async_copy(rows.at[0], acc.at[dv.at[0]],
                                      sem_s).wait()
            plsc.subcore_barrier()
            pltpu.sync_copy(acc.at[pl.ds(s * SPAN, SPAN), :],
                            out_hbm.at[pl.ds(lo + s * SPAN, SPAN), :])
            plsc.subcore_barrier()
            return carry
        lax.fori_loop(0, NCHUNK // 2, chunk_body, 0)

    _process(src_hbm0, packed_hbm0, table_hbm0, out_hbm0)


@functools.partial(
    pl.kernel,
    out_type=jax.ShapeDtypeStruct((LP, HW), F32),
    mesh=_mesh,
    scratch_types=[
        pltpu.VMEM((LPT,), I32),
        pltpu.VMEM((LPT,), I32),
        pltpu.VMEM((128, HW), F32),
        pltpu.SemaphoreType.DMA,
    ],
    compiler_params=_sc_params,
)
def _head(i0_hbm, i1_hbm, gm_hbm, gv_hbm, out_hbm, i0_buf, i1_buf, ra, sem):
    c = lax.axis_index("c")
    s = lax.axis_index("s")
    w = s * 2 + c
    base = w * LPT
    pltpu.sync_copy(i0_hbm.at[pl.ds(base, LPT)], i0_buf)
    pltpu.sync_copy(i1_hbm.at[pl.ds(base, LPT)], i1_buf)

    def body(b, carry):
        pltpu.async_copy(gm_hbm.at[i0_buf.at[pl.ds(b * 128, 128)]],
                         ra, sem).wait()
        pltpu.async_copy(gv_hbm.at[i1_buf.at[pl.ds(b * 128, 128)]],
                         ra, sem, add=True).wait()
        pltpu.sync_copy(ra, out_hbm.at[pl.ds(base + b * 128, 128), :])
        return carry
    lax.fori_loop(0, LPT // 128, body, 0)


def _flag_cols(nrows):
    return (lax.broadcasted_iota(I32, (nrows, 16), 1) == 0).astype(F32)


def _proj_body(x_ref, w_ref, b_ref, o_ref):
    h = jnp.dot(x_ref[...], w_ref[...], preferred_element_type=F32) + b_ref[...]
    o_ref[...] = jnp.concatenate([h, _flag_cols(h.shape[0])], axis=1)


_proj = pl.pallas_call(
    _proj_body,
    grid=(NP // SPAN,),
    in_specs=[pl.BlockSpec((SPAN, H), lambda i: (i, 0)),
              pl.BlockSpec((H, H), lambda i: (0, 0)),
              pl.BlockSpec((1, H), lambda i: (0, 0))],
    out_specs=pl.BlockSpec((SPAN, WA), lambda i: (i, 0)),
    out_shape=jax.ShapeDtypeStruct((NP, WA), F32),
)


def _mean_h(agg_ref, xd_ref, wl_ref, bl_ref, wr_ref):
    aggv = agg_ref[:, :H]
    cnt = agg_ref[:, H:H + 1]
    mean = aggv / jnp.maximum(cnt, 1.0)
    h = (jnp.dot(mean, wl_ref[...], preferred_element_type=F32) + bl_ref[...]
         + jnp.dot(xd_ref[:, :H], wr_ref[...], preferred_element_type=F32))
    return jnp.maximum(h, 0.0)


def _comb1_body(agg_ref, xd_ref, wl_ref, bl_ref, wr_ref, o_ref):
    h = _mean_h(agg_ref, xd_ref, wl_ref, bl_ref, wr_ref)
    o_ref[...] = jnp.concatenate([h, _flag_cols(h.shape[0])], axis=1)


_comb1 = pl.pallas_call(
    _comb1_body,
    grid=(NP // SPAN,),
    in_specs=[pl.BlockSpec((SPAN, WA), lambda i: (i, 0)),
              pl.BlockSpec((SPAN, WA), lambda i: (i, 0)),
              pl.BlockSpec((H, H), lambda i: (0, 0)),
              pl.BlockSpec((1, H), lambda i: (0, 0)),
              pl.BlockSpec((H, H), lambda i: (0, 0))],
    out_specs=pl.BlockSpec((SPAN, WA), lambda i: (i, 0)),
    out_shape=jax.ShapeDtypeStruct((NP, WA), F32),
)


def _comb2_body(agg_ref, xd_ref, wl_ref, bl_ref, wr_ref, wc_ref, bc_ref, o_ref):
    h = _mean_h(agg_ref, xd_ref, wl_ref, bl_ref, wr_ref)
    o_ref[...] = jnp.dot(h, wc_ref[...], preferred_element_type=F32) + bc_ref[...]


_comb2 = pl.pallas_call(
    _comb2_body,
    grid=(NP // SPAN,),
    in_specs=[pl.BlockSpec((SPAN, WA), lambda i: (i, 0)),
              pl.BlockSpec((SPAN, WA), lambda i: (i, 0)),
              pl.BlockSpec((H, H), lambda i: (0, 0)),
              pl.BlockSpec((1, H), lambda i: (0, 0)),
              pl.BlockSpec((H, H), lambda i: (0, 0)),
              pl.BlockSpec((H, HW), lambda i: (0, 0)),
              pl.BlockSpec((1, HW), lambda i: (0, 0))],
    out_specs=pl.BlockSpec((SPAN, HW), lambda i: (i, 0)),
    out_shape=jax.ShapeDtypeStruct((NP, HW), F32),
)


def _pad_edges(ei):
    pad = EP - E
    src = jnp.concatenate([ei[0], jnp.zeros((pad,), I32)])
    dst = jnp.concatenate([ei[1], jnp.full((pad,), 1 << 20, I32)])
    return src, dst


def kernel(x_mouse, x_virus, W_mouse, b_mouse, W_virus, b_virus,
           c1mv_Wl, c1mv_bl, c1mv_Wr, c1vm_Wl, c1vm_bl, c1vm_Wr,
           c2mv_Wl, c2mv_bl, c2mv_Wr, c2vm_Wl, c2vm_bl, c2vm_Wr,
           W_cls, b_cls, edge_index_mv, edge_index_vm, edge_label_index):
    src_mv, dst_mv = _pad_edges(edge_index_mv)
    src_vm, dst_vm = _pad_edges(edge_index_vm)

    pk_mv, pk_vm = _partition(dst_mv, dst_vm)
    hm0 = _proj(x_mouse, W_mouse, b_mouse[None])
    hv0 = _proj(x_virus, W_virus, b_virus[None])

    aggv1 = _aggregate(src_mv, pk_mv, hm0)
    hv1 = _comb1(aggv1, hv0, c1mv_Wl, c1mv_bl[None], c1mv_Wr)
    aggm1 = _aggregate(src_vm, pk_vm, hv0)
    hm1 = _comb1(aggm1, hm0, c1vm_Wl, c1vm_bl[None], c1vm_Wr)

    aggm2 = _aggregate(src_vm, pk_vm, hv1)
    aggv2 = _aggregate(src_mv, pk_mv, hm1)

    wc_m = jnp.pad(W_cls[:H], ((0, 0), (0, HW - 2)))
    wc_v = jnp.pad(W_cls[H:], ((0, 0), (0, HW - 2)))
    bc = jnp.pad(b_cls, (0, HW - 2))[None]
    gm = _comb2(aggm2, hm1, c2vm_Wl, c2vm_bl[None], c2vm_Wr, wc_m, bc)
    gv = _comb2(aggv2, hv1, c2mv_Wl, c2mv_bl[None], c2mv_Wr, wc_v,
                jnp.zeros((1, HW), F32))

    i0 = jnp.concatenate([edge_label_index[0], jnp.zeros((LP - L,), I32)])
    i1 = jnp.concatenate([edge_label_index[1], jnp.zeros((LP - L,), I32)])
    out = _head(i0, i1, gm, gv)
    return out[:L, :2]
